# Initial kernel scaffold; baseline (speedup 1.0000x reference)
import jax
import jax.numpy as jnp
from jax.experimental import pallas as pl


def kernel(x_win, edge_index, W_ih, W_hh, b_ih, b_hh, W1, att_src1, att_dst1, b1, W2, att_src2, att_dst2, b2, W_head, b_head):
    raise NotImplementedError("write your pallas kernel here")


# speedup vs baseline: 1.0055x; 1.0055x over previous
import functools

import jax
import jax.numpy as jnp
from jax.experimental import pallas as pl
from jax.experimental.pallas import tpu as pltpu

_N = 50000
_T = 14
_IN = 16
_LH = 64
_HEADS = 4
_CH = 16

_BN = 2000


def _lstm_body(x_ref, wih_ref, whh_ref, b_ref, h_ref):
    x = x_ref[...]
    wih = wih_ref[...]
    whh = whh_ref[...]
    b = b_ref[...]
    h = jnp.zeros((_BN, _LH), jnp.float32)
    c = jnp.zeros((_BN, _LH), jnp.float32)
    for t in range(_T):
        xt = x[:, t * _IN:(t + 1) * _IN]
        gates = (jnp.dot(xt, wih, preferred_element_type=jnp.float32)
                 + jnp.dot(h, whh, preferred_element_type=jnp.float32) + b)
        i = jax.nn.sigmoid(gates[:, :_LH])
        f = jax.nn.sigmoid(gates[:, _LH:2 * _LH])
        g = jnp.tanh(gates[:, 2 * _LH:3 * _LH])
        o = jax.nn.sigmoid(gates[:, 3 * _LH:])
        c = f * c + i * g
        h = o * jnp.tanh(c)
    h_ref[...] = h


def _lstm(x2d, wih_t, whh_t, b):
    return pl.pallas_call(
        _lstm_body,
        grid=(_N // _BN,),
        in_specs=[
            pl.BlockSpec((_BN, _T * _IN), lambda i: (i, 0)),
            pl.BlockSpec((_IN, 4 * _LH), lambda i: (0, 0)),
            pl.BlockSpec((_LH, 4 * _LH), lambda i: (0, 0)),
            pl.BlockSpec((1, 4 * _LH), lambda i: (0, 0)),
        ],
        out_specs=pl.BlockSpec((_BN, _LH), lambda i: (i, 0)),
        out_shape=jax.ShapeDtypeStruct((_N, _LH), jnp.float32),
    )(x2d, wih_t, whh_t, b)


def _gat_xla(x, src, dst, W, att_src, att_dst, bias):
    n = x.shape[0]
    xp = (x @ W).reshape(n, _HEADS, _CH)
    a_src = jnp.sum(xp * att_src[None], axis=-1)
    a_dst = jnp.sum(xp * att_dst[None], axis=-1)
    alpha = a_src[src] + a_dst[dst]
    alpha = jax.nn.leaky_relu(alpha, negative_slope=0.2)
    amax = jax.ops.segment_max(alpha, dst, num_segments=n)
    amax = jnp.where(jnp.isfinite(amax), amax, 0.0)
    ex = jnp.exp(alpha - amax[dst])
    denom = jax.ops.segment_sum(ex, dst, num_segments=n)
    coef = ex / (denom[dst] + 1e-16)
    msg = xp[src] * coef[:, :, None]
    out = jax.ops.segment_sum(msg, dst, num_segments=n)
    return out.reshape(n, _HEADS * _CH) + bias


def kernel(x_win, edge_index, W_ih, W_hh, b_ih, b_hh, W1, att_src1,
           att_dst1, b1, W2, att_src2, att_dst2, b2, W_head, b_head):
    x2d = x_win.reshape(_N, _T * _IN)
    b = (b_ih + b_hh).reshape(1, 4 * _LH)
    node_embed = _lstm(x2d, W_ih.T, W_hh.T, b)
    loop = jnp.arange(_N, dtype=edge_index.dtype)
    src = jnp.concatenate([edge_index[0], loop])
    dst = jnp.concatenate([edge_index[1], loop])
    h = jax.nn.elu(_gat_xla(node_embed, src, dst, W1, att_src1, att_dst1, b1))
    h = jax.nn.elu(_gat_xla(h, src, dst, W2, att_src2, att_dst2, b2))
    return h @ W_head.T + b_head
